# baseline (speedup 1.0000x reference)
import jax
import jax.numpy as jnp
from jax.experimental import pallas as pl

_NUM_CLASSES = 805
_BLOCK_ROWS = 32


def _onehot_block(x_ref, o_ref):
    x = x_ref[...]
    iota = jax.lax.broadcasted_iota(jnp.int32, o_ref.shape, 2)
    o_ref[...] = (x[:, :, None] == iota).astype(jnp.int32)


def kernel(x):
    n, m = x.shape
    grid = (n // _BLOCK_ROWS,)
    return pl.pallas_call(
        _onehot_block,
        grid=grid,
        in_specs=[pl.BlockSpec((_BLOCK_ROWS, m), lambda i: (i, 0))],
        out_specs=pl.BlockSpec((_BLOCK_ROWS, m, _NUM_CLASSES),
                               lambda i: (i, 0, 0)),
        out_shape=jax.ShapeDtypeStruct((n, m, _NUM_CLASSES), jnp.int32),
    )(x)
